# baseline (speedup 1.0000x reference)
import functools

import jax
import jax.numpy as jnp
from jax import lax
from jax.experimental import pallas as pl
from jax.experimental.pallas import tpu as pltpu
from jax.experimental.pallas import tpu_sc as plsc

VOCAB = 1000000
D_MODEL = 64
BATCH = 4096
SEQ = 200

_NB = BATCH * SEQ
_NW = 32
_BPW = _NB // _NW
_CH = 800
_NCH = _BPW // _CH

@functools.cache
def _build_sc_gather():
    mesh = plsc.VectorSubcoreMesh(core_axis_name="c", subcore_axis_name="s")

    @functools.partial(
        pl.kernel,
        mesh=mesh,
        compiler_params=pltpu.CompilerParams(use_tc_tiling_on_sc=False),
        out_type=jax.ShapeDtypeStruct((_NB, D_MODEL), jnp.float32),
        scratch_types=[
            pltpu.VMEM((_BPW,), jnp.int32),
            pltpu.VMEM((_CH, D_MODEL), jnp.float32),
            pltpu.SemaphoreType.DMA,
        ],
    )
    def _sc_gather(idx_hbm, table_hbm, out_hbm, idx_v, rows_v, sem):
        wid = lax.axis_index("s") * 2 + lax.axis_index("c")
        base = wid * _BPW
        pltpu.sync_copy(idx_hbm.at[pl.ds(base, _BPW)], idx_v)

        def body(g, carry):
            pltpu.async_copy(
                table_hbm.at[idx_v.at[pl.ds(g * _CH, _CH)]], rows_v, sem
            ).wait()
            pltpu.sync_copy(rows_v, out_hbm.at[pl.ds(base + g * _CH, _CH)])
            return carry

        lax.fori_loop(0, _NCH, body, 0)

    return _sc_gather


def kernel(x, table):
    flat = x.reshape(_NB)
    out = _build_sc_gather()(flat, table)
    return out.reshape(BATCH, SEQ, D_MODEL)


# speedup vs baseline: 1.0108x; 1.0108x over previous
import functools

import jax
import jax.numpy as jnp
from jax import lax
from jax.experimental import pallas as pl
from jax.experimental.pallas import tpu as pltpu
from jax.experimental.pallas import tpu_sc as plsc

VOCAB = 1000000
D_MODEL = 64
BATCH = 4096
SEQ = 200

_NB = BATCH * SEQ
_NW = 32
_BPW = _NB // _NW
_CH = 800
_NCH = _BPW // _CH

@functools.cache
def _build_sc_gather():
    mesh = plsc.VectorSubcoreMesh(core_axis_name="c", subcore_axis_name="s")

    @functools.partial(
        pl.kernel,
        mesh=mesh,
        compiler_params=pltpu.CompilerParams(use_tc_tiling_on_sc=False),
        out_type=jax.ShapeDtypeStruct((_NB, D_MODEL), jnp.float32),
        scratch_types=[
            pltpu.VMEM((_BPW,), jnp.int32),
            pltpu.VMEM((_CH, D_MODEL), jnp.float32),
            pltpu.VMEM((_CH, D_MODEL), jnp.float32),
            pltpu.SemaphoreType.DMA,
            pltpu.SemaphoreType.DMA,
        ],
    )
    def _sc_gather(idx_hbm, table_hbm, out_hbm, idx_v, buf0, buf1, sem0, sem1):
        wid = lax.axis_index("s") * 2 + lax.axis_index("c")
        base = wid * _BPW
        pltpu.sync_copy(idx_hbm.at[pl.ds(base, _BPW)], idx_v)

        def gather(g, buf, sem):
            return pltpu.async_copy(
                table_hbm.at[idx_v.at[pl.ds(g * _CH, _CH)]], buf, sem
            )

        def put(g, buf):
            pltpu.sync_copy(buf, out_hbm.at[pl.ds(base + g * _CH, _CH)])

        gather(0, buf0, sem0)

        def body(i, carry):
            g = 2 * i
            gather(g + 1, buf1, sem1)
            pltpu.make_async_copy(
                table_hbm.at[idx_v.at[pl.ds(g * _CH, _CH)]], buf0, sem0
            ).wait()
            put(g, buf0)
            gather(g + 2, buf0, sem0)
            pltpu.make_async_copy(
                table_hbm.at[idx_v.at[pl.ds(g * _CH, _CH)]], buf1, sem1
            ).wait()
            put(g + 1, buf1)
            return carry

        lax.fori_loop(0, _NCH // 2 - 1, body, 0)

        g = _NCH - 2
        gather(g + 1, buf1, sem1)
        pltpu.make_async_copy(
            table_hbm.at[idx_v.at[pl.ds(g * _CH, _CH)]], buf0, sem0
        ).wait()
        put(g, buf0)
        pltpu.make_async_copy(
            table_hbm.at[idx_v.at[pl.ds(g * _CH, _CH)]], buf1, sem1
        ).wait()
        put(g + 1, buf1)

    return _sc_gather


def kernel(x, table):
    flat = x.reshape(_NB)
    out = _build_sc_gather()(flat, table)
    return out.reshape(BATCH, SEQ, D_MODEL)
